# baseline (speedup 1.0000x reference)
import functools

import jax
import jax.numpy as jnp
from jax import lax
from jax.experimental import pallas as pl
from jax.experimental.pallas import tpu as pltpu
from jax.experimental.pallas import tpu_sc as plsc

NC = 2
NS = 16
NW = NC * NS
L = 16
C = 128


def _rel_transform_body(R, nf_ref, w_ref, out_ref):
    nf = nf_ref[...]
    for r in range(R):
        out_ref[r] = jnp.dot(nf, w_ref[r], preferred_element_type=jnp.float32)


def _rel_transform(node_feat, W_rel):
    R, D, _ = W_rel.shape
    N = node_feat.shape[0]
    BN = 1000
    return pl.pallas_call(
        functools.partial(_rel_transform_body, R),
        grid=(N // BN,),
        in_specs=[
            pl.BlockSpec((BN, D), lambda i: (i, 0)),
            pl.BlockSpec((R, D, D), lambda i: (0, 0, 0)),
        ],
        out_specs=pl.BlockSpec((R, BN, D), lambda i: (0, i, 0)),
        out_shape=jax.ShapeDtypeStruct((R, N, D), jnp.float32),
    )(node_feat, W_rel)


def _sc_body(NA, NB, AGG_ROWS, Q, QPT, QPW, D,
             xrel_hbm, gidx_hbm, didx_hbm, qidx_hbm, nf_hbm,
             ga_hbm, gnf_hbm,
             agg_sh, idxg0, idxg1, idxd0, idxd1, rows0, rows1, qv, qv2,
             semi0, semi1, semd0, semd1, semg0, semg1, sem):
    cid = lax.axis_index("c")
    sid = lax.axis_index("s")
    wid = sid * NC + cid
    idxgb = (idxg0, idxg1)
    idxdb = (idxd0, idxd1)
    rowsb = (rows0, rows1)
    semib = (semi0, semi1)
    semdb = (semd0, semd1)
    semgb = (semg0, semg1)
    z = jnp.zeros((L,), jnp.float32)
    nseg = D // L

    def zstore(i, _):
        rows0[i // nseg, pl.ds((i % nseg) * L, L)] = z
        return 0

    lax.fori_loop(0, C * nseg, zstore, 0)

    rows_per_tile = AGG_ROWS // NS
    tb = sid * rows_per_tile
    nz = rows_per_tile // C

    def zfire(j, _):
        pltpu.async_copy(rows0, agg_sh.at[pl.ds(tb + j * C, C)], sem)
        return 0

    lax.fori_loop(0, nz, zfire, 0)

    def zdrain(j, _):
        pltpu.make_async_copy(rows0, agg_sh.at[pl.ds(tb + j * C, C)],
                              sem).wait()
        return 0

    lax.fori_loop(0, nz, zdrain, 0)
    plsc.subcore_barrier()

    CH = jnp.where(cid == 0, NA, NB)
    base = jnp.where(cid == 0, sid * NA, NS * NA + sid * NB)

    def fire_idx(ch, b):
        pltpu.async_copy(gidx_hbm.at[base + ch], idxgb[b], semib[b])
        pltpu.async_copy(didx_hbm.at[base + ch], idxdb[b].at[0], semdb[b])

    def wait_idx(ch, b):
        pltpu.make_async_copy(gidx_hbm.at[base + ch], idxgb[b],
                              semib[b]).wait()
        pltpu.make_async_copy(didx_hbm.at[base + ch], idxdb[b].at[0],
                              semdb[b]).wait()

    def fire_gather(b):
        pltpu.async_copy(xrel_hbm.at[idxgb[b]], rowsb[b], semgb[b])

    def wait_gather(b):
        pltpu.make_async_copy(xrel_hbm.at[idxgb[b]], rowsb[b],
                              semgb[b]).wait()

    fire_idx(0, 0)
    fire_idx(1, 1)
    wait_idx(0, 0)
    fire_gather(0)

    def chunk(ch, b):
        nb = 1 - b
        wait_gather(b)

        @pl.when(ch + 1 < CH)
        def _():
            wait_idx(ch + 1, nb)
            fire_gather(nb)

        pltpu.sync_copy(rowsb[b], agg_sh.at[idxdb[b].at[0]], add=True)

        @pl.when(ch + 2 < CH)
        def _():
            fire_idx(ch + 2, b)

    def chunk2(g, _):
        chunk(2 * g, 0)
        chunk(2 * g + 1, 1)
        return 0

    lax.fori_loop(0, CH // 2, chunk2, 0)
    plsc.subcore_barrier()

    qb = sid * QPT
    pltpu.sync_copy(qidx_hbm.at[pl.ds(qb, QPT)], qv)
    pltpu.async_copy(agg_sh.at[qv], rows1, sem).wait()
    pltpu.sync_copy(rows1, ga_hbm.at[cid, pl.ds(qb, QPT)])

    qb2 = wid * QPW
    pltpu.sync_copy(qidx_hbm.at[pl.ds(qb2, QPW)], qv2)
    pltpu.async_copy(nf_hbm.at[qv2], rows0.at[pl.ds(0, QPW)], sem).wait()
    pltpu.sync_copy(rows0.at[pl.ds(0, QPW)], gnf_hbm.at[pl.ds(qb2, QPW)])


def _sc_aggregate(x_rel, gidx, didx, qidx, node_feat, AGG_ROWS, NA, NB):
    D = node_feat.shape[1]
    Q = qidx.shape[0]
    QPT = Q // NS
    QPW = Q // NW
    mesh = plsc.VectorSubcoreMesh(core_axis_name="c", subcore_axis_name="s",
                                  num_cores=NC, num_subcores=NS)
    body = functools.partial(_sc_body, NA, NB, AGG_ROWS, Q, QPT, QPW, D)
    f = pl.kernel(
        body,
        out_type=[
            jax.ShapeDtypeStruct((NC, Q, D), jnp.float32),
            jax.ShapeDtypeStruct((Q, D), jnp.float32),
        ],
        mesh=mesh,
        compiler_params=pltpu.CompilerParams(needs_layout_passes=False),
        scratch_types=[
            pltpu.VMEM_SHARED((AGG_ROWS, D), jnp.float32),
            pltpu.VMEM((C,), jnp.int32),
            pltpu.VMEM((C,), jnp.int32),
            pltpu.VMEM((1, C), jnp.int32),
            pltpu.VMEM((1, C), jnp.int32),
            pltpu.VMEM((C, D), jnp.float32),
            pltpu.VMEM((C, D), jnp.float32),
            pltpu.VMEM((QPT,), jnp.int32),
            pltpu.VMEM((QPW,), jnp.int32),
            pltpu.SemaphoreType.DMA,
            pltpu.SemaphoreType.DMA,
            pltpu.SemaphoreType.DMA,
            pltpu.SemaphoreType.DMA,
            pltpu.SemaphoreType.DMA,
            pltpu.SemaphoreType.DMA,
            pltpu.SemaphoreType.DMA,
        ],
    )
    return f(x_rel, gidx, didx, qidx, node_feat)


def _head_body(ga_ref, gnf_ref, wr_ref, bg_ref, wf_ref, bf_ref, out_ref):
    D = wr_ref.shape[0]
    Bq = out_ref.shape[0]
    t = (ga_ref[0] + ga_ref[1]
         + jnp.dot(gnf_ref[...], wr_ref[...],
                   preferred_element_type=jnp.float32)
         + bg_ref[...])
    t = jnp.maximum(t, 0.0)
    hid = (jnp.dot(t[:Bq], wf_ref[:D], preferred_element_type=jnp.float32)
           + jnp.dot(t[Bq:], wf_ref[D:], preferred_element_type=jnp.float32)
           + bf_ref[...])
    out_ref[...] = jnp.maximum(hid, 0.0)


def _head(ga, gnf, W_root, b_gnn, W_fc, b_fc):
    B2 = ga.shape[1]
    H = W_fc.shape[1]
    return pl.pallas_call(
        _head_body,
        out_shape=jax.ShapeDtypeStruct((B2 // 2, H), jnp.float32),
    )(ga, gnf, W_root, b_gnn.reshape(1, -1), W_fc, b_fc.reshape(1, -1))


def kernel(x, node_feat, edge_index, edge_type, nest_tensor, food_tensor,
           W_rel, W_root, b_gnn, W_fc, b_fc):
    N, D = node_feat.shape
    R = W_rel.shape[0]
    E = edge_type.shape[0]

    src = edge_index[0].astype(jnp.int32)
    dst = edge_index[1].astype(jnp.int32)
    et = edge_type.astype(jnp.int32)

    x_rel = _rel_transform(node_feat, W_rel).reshape(R * N, D)

    T16 = -(-E // (NS * C))
    NA = max(2, 2 * round(T16 * 0.65 / 2))
    NB = max(2, 2 * (-(-(T16 - NA) // 2)))
    T = NS * (NA + NB)
    pad = T * C - E
    AGG_ROWS = -(-N // (64 * NS)) * (64 * NS)

    gidx = jnp.concatenate([et * N + src,
                            jnp.zeros((pad,), jnp.int32)]).reshape(T, C)
    didx = jnp.concatenate([dst,
                            jnp.full((pad,), AGG_ROWS - 1, jnp.int32)
                            ]).reshape(T, C)
    qidx = jnp.concatenate([nest_tensor.astype(jnp.int32),
                            food_tensor.astype(jnp.int32)])

    ga, gnf = _sc_aggregate(x_rel, gidx, didx, qidx, node_feat,
                            AGG_ROWS, NA, NB)

    return _head(ga, gnf, W_root, b_gnn, W_fc, b_fc)


# speedup vs baseline: 1.3946x; 1.3946x over previous
import functools

import jax
import jax.numpy as jnp
from jax import lax
from jax.experimental import pallas as pl
from jax.experimental.pallas import tpu as pltpu
from jax.experimental.pallas import tpu_sc as plsc

NC = 2
NS = 16
NW = NC * NS
L = 16
C = 128


def _rel_transform_body(R, nf_ref, w_ref, out_ref):
    nf = nf_ref[...]
    for r in range(R):
        out_ref[r] = jnp.dot(nf, w_ref[r], preferred_element_type=jnp.float32)


def _rel_transform(node_feat, W_rel):
    R, D, _ = W_rel.shape
    N = node_feat.shape[0]
    BN = 1000
    return pl.pallas_call(
        functools.partial(_rel_transform_body, R),
        grid=(N // BN,),
        in_specs=[
            pl.BlockSpec((BN, D), lambda i: (i, 0)),
            pl.BlockSpec((R, D, D), lambda i: (0, 0, 0)),
        ],
        out_specs=pl.BlockSpec((R, BN, D), lambda i: (0, i, 0)),
        out_shape=jax.ShapeDtypeStruct((R, N, D), jnp.float32),
    )(node_feat, W_rel)


def _sc_body(NA, NB, AGG_ROWS, QPT, QPW, D,
             xrel_hbm, idx_hbm, qidx_hbm, nf_hbm,
             ga_hbm, gnf_hbm,
             agg_sh, idx0, idx1, rows0, rows1, qv, qv2,
             semi0, semi1, semg0, semg1, sem):
    cid = lax.axis_index("c")
    sid = lax.axis_index("s")
    wid = sid * NC + cid
    idxb = (idx0, idx1)
    rowsb = (rows0, rows1)
    semib = (semi0, semi1)
    semgb = (semg0, semg1)

    z = jnp.zeros((L,), jnp.float32)
    nseg = D // L

    def zstore(i, _):
        rows0[i // nseg, pl.ds((i % nseg) * L, L)] = z
        return 0

    lax.fori_loop(0, C * nseg, zstore, 0)

    rows_per_tile = AGG_ROWS // NS
    tb = sid * rows_per_tile

    def zcopy(j, _):
        pltpu.sync_copy(rows0, agg_sh.at[pl.ds(tb + j * C, C)])
        return 0

    lax.fori_loop(0, rows_per_tile // C, zcopy, 0)
    plsc.subcore_barrier()

    def edge_pipeline(CH, base):
        def fire_idx(ch, b):
            pltpu.async_copy(idx_hbm.at[base + ch], idxb[b], semib[b])

        def wait_idx(ch, b):
            pltpu.make_async_copy(idx_hbm.at[base + ch], idxb[b],
                                  semib[b]).wait()

        def fire_gather(b):
            pltpu.async_copy(xrel_hbm.at[idxb[b].at[0]], rowsb[b], semgb[b])

        def wait_gather(b):
            pltpu.make_async_copy(xrel_hbm.at[idxb[b].at[0]], rowsb[b],
                                  semgb[b]).wait()

        fire_idx(0, 0)
        if CH > 1:
            fire_idx(1, 1)
        wait_idx(0, 0)
        fire_gather(0)

        def chunk(ch, b):
            nb = 1 - b
            wait_gather(b)

            @pl.when(ch + 1 < CH)
            def _():
                wait_idx(ch + 1, nb)
                fire_gather(nb)

            pltpu.sync_copy(rowsb[b], agg_sh.at[idxb[b].at[1]], add=True)

            @pl.when(ch + 2 < CH)
            def _():
                fire_idx(ch + 2, b)

        def chunk2(g, _):
            chunk(2 * g, 0)
            chunk(2 * g + 1, 1)
            return 0

        lax.fori_loop(0, CH // 2, chunk2, 0)
        if CH % 2:
            chunk(CH - 1, (CH - 1) % 2)

    @pl.when(cid == 0)
    def _():
        edge_pipeline(NA, sid * NA)

    @pl.when(cid == 1)
    def _():
        edge_pipeline(NB, NS * NA + sid * NB)

    plsc.subcore_barrier()

    qb = sid * QPT
    pltpu.sync_copy(qidx_hbm.at[pl.ds(qb, QPT)], qv)
    pltpu.async_copy(agg_sh.at[qv], rows1, sem).wait()
    pltpu.sync_copy(rows1, ga_hbm.at[cid, pl.ds(qb, QPT)])

    qb2 = wid * QPW
    pltpu.sync_copy(qidx_hbm.at[pl.ds(qb2, QPW)], qv2)
    pltpu.async_copy(nf_hbm.at[qv2], rows0.at[pl.ds(0, QPW)], sem).wait()
    pltpu.sync_copy(rows0.at[pl.ds(0, QPW)], gnf_hbm.at[pl.ds(qb2, QPW)])


def _sc_aggregate(x_rel, idx, qidx, node_feat, AGG_ROWS, NA, NB):
    D = node_feat.shape[1]
    Q = qidx.shape[0]
    QPT = Q // NS
    QPW = Q // NW
    mesh = plsc.VectorSubcoreMesh(core_axis_name="c", subcore_axis_name="s",
                                  num_cores=NC, num_subcores=NS)
    body = functools.partial(_sc_body, NA, NB, AGG_ROWS, QPT, QPW, D)
    f = pl.kernel(
        body,
        out_type=[
            jax.ShapeDtypeStruct((NC, Q, D), jnp.float32),
            jax.ShapeDtypeStruct((Q, D), jnp.float32),
        ],
        mesh=mesh,
        scratch_types=[
            pltpu.VMEM_SHARED((AGG_ROWS, D), jnp.float32),
            pltpu.VMEM((2, C), jnp.int32),
            pltpu.VMEM((2, C), jnp.int32),
            pltpu.VMEM((C, D), jnp.float32),
            pltpu.VMEM((C, D), jnp.float32),
            pltpu.VMEM((QPT,), jnp.int32),
            pltpu.VMEM((QPW,), jnp.int32),
            pltpu.SemaphoreType.DMA,
            pltpu.SemaphoreType.DMA,
            pltpu.SemaphoreType.DMA,
            pltpu.SemaphoreType.DMA,
            pltpu.SemaphoreType.DMA,
        ],
    )
    return f(x_rel, idx, qidx, node_feat)


def _head_body(ga_ref, gnf_ref, wr_ref, bg_ref, wf_ref, bf_ref, out_ref):
    D = wr_ref.shape[0]
    Bq = out_ref.shape[0]
    t = (ga_ref[0] + ga_ref[1]
         + jnp.dot(gnf_ref[...], wr_ref[...],
                   preferred_element_type=jnp.float32)
         + bg_ref[...])
    t = jnp.maximum(t, 0.0)
    hid = (jnp.dot(t[:Bq], wf_ref[:D], preferred_element_type=jnp.float32)
           + jnp.dot(t[Bq:], wf_ref[D:], preferred_element_type=jnp.float32)
           + bf_ref[...])
    out_ref[...] = jnp.maximum(hid, 0.0)


def _head(ga, gnf, W_root, b_gnn, W_fc, b_fc):
    B2 = ga.shape[1]
    H = W_fc.shape[1]
    return pl.pallas_call(
        _head_body,
        out_shape=jax.ShapeDtypeStruct((B2 // 2, H), jnp.float32),
    )(ga, gnf, W_root, b_gnn.reshape(1, -1), W_fc, b_fc.reshape(1, -1))


def kernel(x, node_feat, edge_index, edge_type, nest_tensor, food_tensor,
           W_rel, W_root, b_gnn, W_fc, b_fc):
    N, D = node_feat.shape
    R = W_rel.shape[0]
    E = edge_type.shape[0]

    src = edge_index[0].astype(jnp.int32)
    dst = edge_index[1].astype(jnp.int32)
    et = edge_type.astype(jnp.int32)

    x_rel = _rel_transform(node_feat, W_rel).reshape(R * N, D)

    T16 = -(-E // (NS * C))
    NA = max(1, min(T16 - 1, round(T16 * 0.65)))
    NB = T16 - NA
    T = NS * (NA + NB)
    pad = T * C - E
    AGG_ROWS = -(-N // (64 * NS)) * (64 * NS)

    gidx = jnp.concatenate([et * N + src,
                            jnp.zeros((pad,), jnp.int32)]).reshape(T, C)
    didx = jnp.concatenate([dst,
                            jnp.full((pad,), AGG_ROWS - 1, jnp.int32)
                            ]).reshape(T, C)
    idx = jnp.stack([gidx, didx], axis=1)
    qidx = jnp.concatenate([nest_tensor.astype(jnp.int32),
                            food_tensor.astype(jnp.int32)])

    ga, gnf = _sc_aggregate(x_rel, idx, qidx, node_feat, AGG_ROWS, NA, NB)

    return _head(ga, gnf, W_root, b_gnn, W_fc, b_fc)
